# baseline (speedup 1.0000x reference)
import functools

import jax
import jax.numpy as jnp
from jax import lax
from jax.experimental import pallas as pl
from jax.experimental.pallas import tpu as pltpu
from jax.experimental.pallas import tpu_sc as plsc

WORD_NUM = 1000000
EMBED = 64
HIDDEN = 64
LABELS = 128
BATCH = 4096
SEQ = 200
HALF = SEQ // 2

_INFO = plsc.get_sparse_core_info()
NC = _INFO.num_cores
NS = _INFO.num_subcores
NW = NC * NS
SPT = BATCH // NW
NLANE = EMBED // 16


CH0 = 104
CH1 = 96


def _pool_body(x_hbm, emb_hbm, out_hbm, idx_v, rows_v, acc_v, sem0, sem1):
    wid = lax.axis_index("s") * NC + lax.axis_index("c")
    pltpu.sync_copy(x_hbm.at[pl.ds(wid * SPT * SEQ, SPT * SEQ)], idx_v)

    sems = (sem0, sem1)

    def issue(s, b):
        pltpu.async_copy(
            emb_hbm.at[idx_v.at[pl.ds(s * SEQ, CH0)]],
            rows_v.at[b, pl.ds(0, CH0)],
            sems[b],
        )
        pltpu.async_copy(
            emb_hbm.at[idx_v.at[pl.ds(s * SEQ + CH0, CH1)]],
            rows_v.at[b, pl.ds(CH0, CH1)],
            sems[b],
        )

    def drain(b):
        pltpu.make_async_copy(
            emb_hbm.at[idx_v.at[pl.ds(0, CH0)]],
            rows_v.at[b, pl.ds(0, CH0)],
            sems[b],
        ).wait()
        pltpu.make_async_copy(
            emb_hbm.at[idx_v.at[pl.ds(CH0, CH1)]],
            rows_v.at[b, pl.ds(CH0, CH1)],
            sems[b],
        ).wait()

    zero = jnp.zeros((16,), jnp.float32)

    def reduce_store(s, b):
        def red(r, accs):
            return tuple(
                accs[j] + rows_v[b, r, pl.ds(16 * j, 16)] for j in range(NLANE)
            )

        accs = lax.fori_loop(0, SEQ, red, (zero,) * NLANE)
        for j in range(NLANE):
            acc_v[pl.ds(s * EMBED + 16 * j, 16)] = accs[j]

    issue(0, 0)
    issue(1, 1)

    def pair_body(p, carry):
        s0 = 2 * p
        for b in range(2):
            s = s0 + b
            drain(b)
            reduce_store(s, b)

            @pl.when(s + 2 < SPT)
            def _():
                issue(s + 2, b)

        return carry

    lax.fori_loop(0, SPT // 2, pair_body, 0)
    pltpu.sync_copy(acc_v, out_hbm.at[pl.ds(wid * SPT * EMBED, SPT * EMBED)])


@functools.partial(jax.jit, static_argnames=())
def _pool(x, emb):
    mesh = plsc.VectorSubcoreMesh(core_axis_name="c", subcore_axis_name="s")
    return pl.kernel(
        _pool_body,
        out_type=jax.ShapeDtypeStruct((BATCH * EMBED,), jnp.float32),
        mesh=mesh,
        compiler_params=pltpu.CompilerParams(use_tc_tiling_on_sc=False),
        scratch_types=[
            pltpu.VMEM((SPT * SEQ,), jnp.int32),
            pltpu.VMEM((2, SEQ, EMBED), jnp.float32),
            pltpu.VMEM((SPT * EMBED,), jnp.float32),
            pltpu.SemaphoreType.DMA,
            pltpu.SemaphoreType.DMA,
        ],
    )(x, emb)


def _mlp_body(p_ref, wh_ref, bh_ref, wo_ref, bo_ref, o_ref):
    p = p_ref[...] * (1.0 / SEQ)
    h = jnp.dot(p, wh_ref[...], preferred_element_type=jnp.float32) + bh_ref[...]
    o_ref[...] = (
        jnp.dot(h, wo_ref[...], preferred_element_type=jnp.float32) + bo_ref[...]
    )


def _mlp(pooled, W_h, b_h, W_o, b_o):
    bb = 1024
    return pl.pallas_call(
        _mlp_body,
        grid=(BATCH // bb,),
        in_specs=[
            pl.BlockSpec((bb, EMBED), lambda i: (i, 0)),
            pl.BlockSpec((EMBED, HIDDEN), lambda i: (0, 0)),
            pl.BlockSpec((1, HIDDEN), lambda i: (0, 0)),
            pl.BlockSpec((HIDDEN, LABELS), lambda i: (0, 0)),
            pl.BlockSpec((1, LABELS), lambda i: (0, 0)),
        ],
        out_specs=pl.BlockSpec((bb, LABELS), lambda i: (i, 0)),
        out_shape=jax.ShapeDtypeStruct((BATCH, LABELS), jnp.float32),
    )(pooled, W_h, b_h.reshape(1, HIDDEN), W_o, b_o.reshape(1, LABELS))


def kernel(X, emb, W_h, b_h, W_o, b_o):
    pooled = _pool(X.astype(jnp.int32).reshape(-1), emb).reshape(BATCH, EMBED)
    return _mlp(pooled, W_h, b_h, W_o, b_o)


# speedup vs baseline: 1.8702x; 1.8702x over previous
import functools

import jax
import jax.numpy as jnp
from jax import lax
from jax.experimental import pallas as pl
from jax.experimental.pallas import tpu as pltpu
from jax.experimental.pallas import tpu_sc as plsc

WORD_NUM = 1000000
EMBED = 64
HIDDEN = 64
LABELS = 128
BATCH = 4096
SEQ = 200

_INFO = plsc.get_sparse_core_info()
NC = _INFO.num_cores
NS = _INFO.num_subcores
NW = NC * NS
SPT = BATCH // NW
HWORDS = LABELS // 2

CH0 = 104
CH1 = 96


def _fold_body(wh_ref, bh_ref, wo_ref, bo_ref, wf_ref, bf_ref):
    wf = jnp.dot(wh_ref[...], wo_ref[...], preferred_element_type=jnp.float32)
    wf_ref[...] = wf * (1.0 / SEQ)
    bf_ref[...] = (
        jnp.dot(bh_ref[...], wo_ref[...], preferred_element_type=jnp.float32)
        + bo_ref[...]
    )


def _fold(W_h, b_h, W_o, b_o):
    return pl.pallas_call(
        _fold_body,
        out_shape=(
            jax.ShapeDtypeStruct((EMBED, LABELS), jnp.float32),
            jax.ShapeDtypeStruct((1, LABELS), jnp.float32),
        ),
    )(W_h, b_h.reshape(1, HIDDEN), W_o, b_o.reshape(1, LABELS))


_VBLK = 8192
_VBLK2 = _VBLK // 2
_NBLK = -(-WORD_NUM // _VBLK)
_TROWS = _NBLK * _VBLK2
_QROWS = 2 * _TROWS


def _bf16_bits(x):
    b = lax.bitcast_convert_type(x.astype(jnp.bfloat16), jnp.uint16)
    return b.astype(jnp.uint32)


def _table_body(embt_ref, wf_ref, o_ref):
    wf = wf_ref[...].astype(jnp.bfloat16)
    embt = embt_ref[...].astype(jnp.bfloat16)

    def words(cols):
        lo = lax.dot_general(
            cols, wf[:, :HWORDS],
            (((0,), (0,)), ((), ())),
            preferred_element_type=jnp.float32,
        )
        hi = lax.dot_general(
            cols, wf[:, HWORDS:],
            (((0,), (0,)), ((), ())),
            preferred_element_type=jnp.float32,
        )
        return _bf16_bits(lo) | (_bf16_bits(hi) << 16)

    w = jnp.concatenate(
        [words(embt[:, :_VBLK2]), words(embt[:, _VBLK2:])], axis=1
    )
    o_ref[...] = lax.bitcast_convert_type(w, jnp.int32)


def _table(embt, wf):
    return pl.pallas_call(
        _table_body,
        grid=(_NBLK,),
        in_specs=[
            pl.BlockSpec((EMBED, _VBLK), lambda i: (0, i)),
            pl.BlockSpec((EMBED, LABELS), lambda i: (0, 0)),
        ],
        out_specs=pl.BlockSpec((_VBLK2, LABELS), lambda i: (i, 0)),
        out_shape=jax.ShapeDtypeStruct((_TROWS, LABELS), jnp.int32),
    )(embt, wf)


def _pool_body(x_hbm, tab_hbm, bf_hbm, out_hbm, idx_v, rows_v, acc_v, bf_v, sem0, sem1):
    wid = lax.axis_index("s") * NC + lax.axis_index("c")
    pltpu.sync_copy(x_hbm.at[pl.ds(wid * SPT * SEQ, SPT * SEQ)], idx_v)
    pltpu.sync_copy(bf_hbm, bf_v)

    sems = (sem0, sem1)

    def issue(s, b):
        pltpu.async_copy(
            tab_hbm.at[idx_v.at[pl.ds(s * SEQ, CH0)]],
            rows_v.at[b, pl.ds(0, CH0)],
            sems[b],
        )
        pltpu.async_copy(
            tab_hbm.at[idx_v.at[pl.ds(s * SEQ + CH0, CH1)]],
            rows_v.at[b, pl.ds(CH0, CH1)],
            sems[b],
        )

    def drain(b):
        pltpu.make_async_copy(
            tab_hbm.at[idx_v.at[pl.ds(0, CH0)]],
            rows_v.at[b, pl.ds(0, CH0)],
            sems[b],
        ).wait()
        pltpu.make_async_copy(
            tab_hbm.at[idx_v.at[pl.ds(CH0, CH1)]],
            rows_v.at[b, pl.ds(CH0, CH1)],
            sems[b],
        ).wait()

    zero = jnp.zeros((16,), jnp.float32)
    bias = [bf_v[pl.ds(16 * m, 16)] for m in range(LABELS // 16)]
    nk = HWORDS // 16

    def reduce_store(s, b):
        def red(r, accs):
            out = list(accs)
            for k in range(nk):
                w = rows_v[b, r, pl.ds(16 * k, 16)]
                e, o = plsc.unpack(
                    plsc.bitcast(w, jnp.bfloat16),
                    format=plsc.PackFormat.INTERLEAVED,
                )
                out[k] = out[k] + e
                out[nk + k] = out[nk + k] + o
            return tuple(out)

        accs = lax.fori_loop(0, SEQ, red, (zero,) * (LABELS // 16))
        for m in range(LABELS // 16):
            acc_v[pl.ds(s * LABELS + 16 * m, 16)] = accs[m] + bias[m]

    issue(0, 0)
    issue(1, 1)

    def pair_body(p, carry):
        s0 = 2 * p
        for b in range(2):
            s = s0 + b
            drain(b)
            reduce_store(s, b)

            @pl.when(s + 2 < SPT)
            def _():
                issue(s + 2, b)

        return carry

    lax.fori_loop(0, SPT // 2, pair_body, 0)
    pltpu.sync_copy(acc_v, out_hbm.at[pl.ds(wid * SPT * LABELS, SPT * LABELS)])


@functools.partial(jax.jit, static_argnames=())
def _pool(x, tab, bf):
    mesh = plsc.VectorSubcoreMesh(core_axis_name="c", subcore_axis_name="s")
    return pl.kernel(
        _pool_body,
        out_type=jax.ShapeDtypeStruct((BATCH * LABELS,), jnp.float32),
        mesh=mesh,
        compiler_params=pltpu.CompilerParams(
            use_tc_tiling_on_sc=False, needs_layout_passes=False
        ),
        scratch_types=[
            pltpu.VMEM((SPT * SEQ,), jnp.int32),
            pltpu.VMEM((2, SEQ, HWORDS), jnp.int32),
            pltpu.VMEM((SPT * LABELS,), jnp.float32),
            pltpu.VMEM((LABELS,), jnp.float32),
            pltpu.SemaphoreType.DMA,
            pltpu.SemaphoreType.DMA,
        ],
    )(x, tab, bf)


def kernel(X, emb, W_h, b_h, W_o, b_o):
    wf, bf = _fold(W_h, b_h, W_o, b_o)
    tab = _table(emb.T, wf).reshape(_QROWS, HWORDS)
    v = X.astype(jnp.int32)
    q = ((v >> 13) << 13) + ((v & (_VBLK2 - 1)) << 1) + ((v & (_VBLK - 1)) >> 12)
    out = _pool(q.reshape(-1), tab, bf.reshape(-1))
    return out.reshape(BATCH, LABELS)


# speedup vs baseline: 2.4282x; 1.2984x over previous
import functools

import jax
import jax.numpy as jnp
from jax import lax
from jax.experimental import pallas as pl
from jax.experimental.pallas import tpu as pltpu
from jax.experimental.pallas import tpu_sc as plsc

WORD_NUM = 1000000
EMBED = 64
HIDDEN = 64
LABELS = 128
BATCH = 4096
SEQ = 200

_INFO = plsc.get_sparse_core_info()
NC = _INFO.num_cores
NS = _INFO.num_subcores
NW = NC * NS
SPT = BATCH // NW
HWORDS = LABELS // 2

CH0 = 104
CH1 = 96


def _fold_body(wh_ref, bh_ref, wo_ref, bo_ref, wf_ref, bf_ref):
    wf = jnp.dot(wh_ref[...], wo_ref[...], preferred_element_type=jnp.float32)
    wf_ref[...] = wf * (1.0 / SEQ)
    bf_ref[...] = (
        jnp.dot(bh_ref[...], wo_ref[...], preferred_element_type=jnp.float32)
        + bo_ref[...]
    )


def _fold(W_h, b_h, W_o, b_o):
    return pl.pallas_call(
        _fold_body,
        out_shape=(
            jax.ShapeDtypeStruct((EMBED, LABELS), jnp.float32),
            jax.ShapeDtypeStruct((1, LABELS), jnp.float32),
        ),
    )(W_h, b_h.reshape(1, HIDDEN), W_o, b_o.reshape(1, LABELS))


_VBLK = 32768
_VBLK2 = _VBLK // 2
_NBLK = -(-WORD_NUM // _VBLK)
_TROWS = _NBLK * _VBLK2
_QROWS = 2 * _TROWS


def _bf16_bits(x):
    b = lax.bitcast_convert_type(x.astype(jnp.bfloat16), jnp.uint16)
    return b.astype(jnp.uint32)


def _table_body(embt_ref, wf_ref, o_ref):
    wf = wf_ref[...].astype(jnp.bfloat16)
    embt = embt_ref[...].astype(jnp.bfloat16)

    def words(cols):
        lo = lax.dot_general(
            cols, wf[:, :HWORDS],
            (((0,), (0,)), ((), ())),
            preferred_element_type=jnp.float32,
        )
        hi = lax.dot_general(
            cols, wf[:, HWORDS:],
            (((0,), (0,)), ((), ())),
            preferred_element_type=jnp.float32,
        )
        return _bf16_bits(lo) | (_bf16_bits(hi) << 16)

    w = jnp.concatenate(
        [words(embt[:, :_VBLK2]), words(embt[:, _VBLK2:])], axis=1
    )
    o_ref[...] = lax.bitcast_convert_type(w, jnp.int32)


def _table(embt, wf):
    return pl.pallas_call(
        _table_body,
        grid=(_NBLK,),
        in_specs=[
            pl.BlockSpec((EMBED, _VBLK), lambda i: (0, i)),
            pl.BlockSpec((EMBED, LABELS), lambda i: (0, 0)),
        ],
        out_specs=pl.BlockSpec((_VBLK2, LABELS), lambda i: (i, 0)),
        out_shape=jax.ShapeDtypeStruct((_TROWS, LABELS), jnp.int32),
    )(embt, wf)


def _pool_body(x_hbm, tab_hbm, bf_hbm, out_hbm, idx_v, rows_v, acc_v, bf_v, sem0, sem1, sem2, sem3):
    wid = lax.axis_index("s") * NC + lax.axis_index("c")
    pltpu.sync_copy(x_hbm.at[pl.ds(wid * SPT * SEQ, SPT * SEQ)], idx_v)
    pltpu.sync_copy(bf_hbm, bf_v)

    sems = (sem0, sem1, sem2, sem3)
    nbuf = len(sems)

    def issue(s, b):
        pltpu.async_copy(
            tab_hbm.at[idx_v.at[pl.ds(s * SEQ, CH0)]],
            rows_v.at[b, pl.ds(0, CH0)],
            sems[b],
        )
        pltpu.async_copy(
            tab_hbm.at[idx_v.at[pl.ds(s * SEQ + CH0, CH1)]],
            rows_v.at[b, pl.ds(CH0, CH1)],
            sems[b],
        )

    def drain(b):
        pltpu.make_async_copy(
            tab_hbm.at[idx_v.at[pl.ds(0, CH0)]],
            rows_v.at[b, pl.ds(0, CH0)],
            sems[b],
        ).wait()
        pltpu.make_async_copy(
            tab_hbm.at[idx_v.at[pl.ds(CH0, CH1)]],
            rows_v.at[b, pl.ds(CH0, CH1)],
            sems[b],
        ).wait()

    zero = jnp.zeros((16,), jnp.float32)
    bias = [bf_v[pl.ds(16 * m, 16)] for m in range(LABELS // 16)]
    nk = HWORDS // 16

    def reduce_store(s, b):
        def red(r, accs):
            out = list(accs)
            for k in range(nk):
                w = rows_v[b, r, pl.ds(16 * k, 16)]
                e, o = plsc.unpack(
                    plsc.bitcast(w, jnp.bfloat16),
                    format=plsc.PackFormat.INTERLEAVED,
                )
                out[k] = out[k] + e
                out[nk + k] = out[nk + k] + o
            return tuple(out)

        accs = lax.fori_loop(0, SEQ, red, (zero,) * (LABELS // 16))
        for m in range(LABELS // 16):
            acc_v[pl.ds(s * LABELS + 16 * m, 16)] = accs[m] + bias[m]

    for b in range(4):
        issue(b, b)

    def group_body(p, carry):
        s0 = nbuf * p
        for b in range(nbuf):
            s = s0 + b
            drain(b)
            reduce_store(s, b)

            @pl.when(s + nbuf < SPT)
            def _():
                issue(s + nbuf, b)

        return carry

    lax.fori_loop(0, SPT // nbuf, group_body, 0)
    pltpu.sync_copy(acc_v, out_hbm.at[pl.ds(wid * SPT * LABELS, SPT * LABELS)])


@functools.partial(jax.jit, static_argnames=())
def _pool(x, tab, bf):
    mesh = plsc.VectorSubcoreMesh(core_axis_name="c", subcore_axis_name="s")
    return pl.kernel(
        _pool_body,
        out_type=jax.ShapeDtypeStruct((BATCH * LABELS,), jnp.float32),
        mesh=mesh,
        compiler_params=pltpu.CompilerParams(
            use_tc_tiling_on_sc=False, needs_layout_passes=False
        ),
        scratch_types=[
            pltpu.VMEM((SPT * SEQ,), jnp.int32),
            pltpu.VMEM((4, SEQ, HWORDS), jnp.int32),
            pltpu.VMEM((SPT * LABELS,), jnp.float32),
            pltpu.VMEM((LABELS,), jnp.float32),
            pltpu.SemaphoreType.DMA,
            pltpu.SemaphoreType.DMA,
            pltpu.SemaphoreType.DMA,
            pltpu.SemaphoreType.DMA,
        ],
    )(x, tab, bf)


def kernel(X, emb, W_h, b_h, W_o, b_o):
    wf, bf = _fold(W_h, b_h, W_o, b_o)
    tab = _table(emb.T, wf).reshape(_QROWS, HWORDS)
    v = X.astype(jnp.int32)
    q = ((v >> 15) << 15) + ((v & (_VBLK2 - 1)) << 1) + ((v & (_VBLK - 1)) >> 14)
    out = _pool(q.reshape(-1), tab, bf.reshape(-1))
    return out.reshape(BATCH, LABELS)
